# Initial kernel scaffold; baseline (speedup 1.0000x reference)
import jax
import jax.numpy as jnp
from jax.experimental import pallas as pl


def kernel(X, T, C, D, S, mask, idx_ij, idx_ijF, idx_ijT, idx_iT, b0, b1, b2, lam, eta, nu, mu, kap, b):
    raise NotImplementedError("write your pallas kernel here")


# speedup vs baseline: 203.5436x; 203.5436x over previous
import functools

import jax
import jax.numpy as jnp
from jax import lax
from jax.experimental import pallas as pl
from jax.experimental.pallas import tpu as pltpu
from jax.experimental.pallas import tpu_sc as plsc

N = 256
KDEG = 4
NCOMB = 4
NT = 3
NCI = 10
B2 = 2
EPS = 1e-16
F32 = jnp.float32
I32 = jnp.int32


def _tc_body(x_ref, c_ref, d_ref, s_ref, m_ref, scal_ref, kap_ref, b_ref,
             xx_out, tmp0_out, base_out, maskb_out, elbd_out, exps_out):
    X = x_ref[...]
    b0 = scal_ref[0:1, 0:1]
    b1 = scal_ref[0:1, 1:2]
    lam = scal_ref[0:1, 3:4]
    eta = scal_ref[0:1, 4:5]
    nu = scal_ref[0:1, 5:6]
    mu = scal_ref[0:1, 6:7]
    Ts = scal_ref[0:1, 7:8]
    bb = b_ref[...]

    p_i = lax.broadcasted_iota(I32, (N * B2, N), 0)
    j_i = lax.broadcasted_iota(I32, (N * B2, N), 1)
    sel = (p_i == B2 * j_i + bb).astype(F32)
    hp = lax.Precision.HIGHEST
    C_b = jnp.dot(c_ref[...], sel, preferred_element_type=F32, precision=hp)
    D_b = jnp.dot(d_ref[...], sel, preferred_element_type=F32, precision=hp)
    S_b = jnp.dot(s_ref[...], sel, preferred_element_type=F32, precision=hp)
    M_b = jnp.dot(m_ref[...], sel, preferred_element_type=F32, precision=hp)

    r_i = lax.broadcasted_iota(I32, (N, N), 0)
    c_i = lax.broadcasted_iota(I32, (N, N), 1)
    eye = (r_i == c_i).astype(F32)
    XT = lax.dot_general(X, eye, (((0,), (0,)), ((), ())),
                         preferred_element_type=F32, precision=hp)

    X_ = 1.0 - X + EPS
    lX_ = jnp.log(X_)
    srow = jnp.sum(lX_, axis=1, keepdims=True)
    tmp0 = jnp.exp(srow - lX_)
    XX_ = X / X_
    exps = jnp.exp(srow)

    coup = lam + eta * D_b + nu * C_b + mu * S_b
    base = (4.0 * XT - 2.0) * coup + 100.0 * (M_b - 1.0)

    e1a = jnp.sum((b0 + jnp.sum(XX_, axis=1, keepdims=True) * b1) * exps)
    e2 = jnp.sum(coup * (1.0 + 4.0 * X * XT - 2.0 * (X + XT)) * M_b)
    e4 = -jnp.sum(Ts) * jnp.sum((X * jnp.log(X + EPS) + X_ * lX_) * M_b)

    xx_out[...] = XX_
    tmp0_out[...] = tmp0
    base_out[...] = base
    maskb_out[...] = M_b
    elbd_out[...] = jnp.reshape(e1a + e2 + e4, (1, 1))
    exps_out[...] = jnp.reshape(exps, (1, N))


def _tc_prep(X, C2, D2, S2, M2, scal, kap, b2d):
    return pl.pallas_call(
        _tc_body,
        out_shape=[
            jax.ShapeDtypeStruct((N, N), F32),
            jax.ShapeDtypeStruct((N, N), F32),
            jax.ShapeDtypeStruct((N, N), F32),
            jax.ShapeDtypeStruct((N, N), F32),
            jax.ShapeDtypeStruct((1, 1), F32),
            jax.ShapeDtypeStruct((1, N), F32),
        ],
    )(X, C2, D2, S2, M2, scal, kap, b2d)


def _splat(v):
    return jnp.broadcast_to(jnp.asarray(v, I32), (16,))


def _sc_make(nc, nw, rpw):
    mesh = plsc.VectorSubcoreMesh(core_axis_name="c", subcore_axis_name="s")

    @functools.partial(
        pl.kernel,
        out_type=[
            jax.ShapeDtypeStruct((N, N), F32),
            jax.ShapeDtypeStruct((nw, 2, 16), F32),
        ],
        mesh=mesh,
        compiler_params=pltpu.CompilerParams(needs_layout_passes=False),
        scratch_types=[
            pltpu.VMEM((rpw, N), F32),
            pltpu.VMEM((rpw, N), F32),
            pltpu.VMEM((rpw, N), F32),
            pltpu.VMEM((rpw, N), F32),
            pltpu.VMEM((rpw, N), F32),
            pltpu.VMEM((rpw,), F32),
            pltpu.VMEM((16,), F32),
            pltpu.VMEM((16,), I32),
            pltpu.VMEM((rpw, N * KDEG * B2), I32),
            pltpu.VMEM((rpw, N * NCOMB * B2), I32),
            pltpu.VMEM((rpw, N * NCOMB * NT * B2), I32),
            pltpu.VMEM((rpw, NCI * 2 * B2), I32),
            pltpu.VMEM((rpw, N), F32),
            pltpu.VMEM((2, 16), F32),
        ],
    )
    def sc_kernel(xx_hbm, tmp0_hbm, base_hbm, maskb_hbm, x_hbm, exps_hbm,
                  pf_hbm, pi_hbm, ij_hbm, ijF_hbm, ijT_hbm, iT_hbm,
                  sig_out, part_out,
                  xx_s, tmp0_s, base_s, mask_s, x_s, exps_s, pf_s, pi_s,
                  ij_s, ijF_s, ijT_s, iT_s, out_s, part_s):
        wid = lax.axis_index("s") * nc + lax.axis_index("c")
        base_row = wid * rpw

        pltpu.sync_copy(xx_hbm.at[pl.ds(base_row, rpw)], xx_s)
        pltpu.sync_copy(tmp0_hbm.at[pl.ds(base_row, rpw)], tmp0_s)
        pltpu.sync_copy(base_hbm.at[pl.ds(base_row, rpw)], base_s)
        pltpu.sync_copy(maskb_hbm.at[pl.ds(base_row, rpw)], mask_s)
        pltpu.sync_copy(x_hbm.at[pl.ds(base_row, rpw)], x_s)
        pltpu.sync_copy(exps_hbm.at[wid], exps_s)
        pltpu.sync_copy(pf_hbm, pf_s)
        pltpu.sync_copy(pi_hbm, pi_s)
        pltpu.sync_copy(ij_hbm.at[pl.ds(base_row, rpw)], ij_s)
        pltpu.sync_copy(ijF_hbm.at[pl.ds(base_row, rpw)], ijF_s)
        pltpu.sync_copy(ijT_hbm.at[pl.ds(base_row, rpw)], ijT_s)
        pltpu.sync_copy(iT_hbm.at[pl.ds(base_row, rpw)], iT_s)

        zero16 = _splat(0)
        lane = lax.broadcasted_iota(I32, (16,), 0)
        bvec = pi_s[...]
        P0 = plsc.load_gather(pf_s, [zero16])
        P1 = plsc.load_gather(pf_s, [_splat(1)])
        P2 = plsc.load_gather(pf_s, [_splat(2)])
        kapv = [plsc.load_gather(pf_s, [_splat(3 + k)]) for k in range(KDEG)]
        Kv = plsc.load_gather(pf_s, [_splat(7)])
        invTs = plsc.load_gather(pf_s, [_splat(8)])
        b2v = plsc.load_gather(pf_s, [_splat(9)])

        def chunk_body(t, e3acc):
            r = t // 16
            a0 = (t % 16) * 16
            rsplat = jnp.broadcast_to(r, (16,))
            aidx = a0 + lane

            aKB = aidx * (KDEG * B2) + bvec
            aFB = aidx * (NCOMB * B2) + bvec
            aTB = aidx * (NCOMB * NT * B2) + bvec

            s0 = jnp.zeros((16,), F32)
            s2 = jnp.zeros((16,), F32)
            for k in range(KDEG):
                colv = plsc.load_gather(ij_s, [rsplat, aKB + k * B2])
                s0 = s0 + plsc.load_gather(xx_s, [rsplat, colv])
                gx = plsc.load_gather(x_s, [rsplat, colv])
                s2 = s2 + (4.0 * gx - 2.0) * kapv[k]

            s1 = jnp.zeros((16,), F32)
            for c in range(NCOMB):
                colF = plsc.load_gather(ijF_s, [rsplat, aFB + c * B2])
                gF = plsc.load_gather(xx_s, [rsplat, colF])
                tsum = jnp.zeros((16,), F32)
                for t_ in range(NT):
                    colT = plsc.load_gather(
                        ijT_s, [rsplat, aTB + (c * NT + t_) * B2])
                    tsum = tsum + plsc.load_gather(xx_s, [rsplat, colT])
                s1 = s1 + gF * tsum

            tmp0v = plsc.load_gather(tmp0_s, [rsplat, aidx])
            basev = plsc.load_gather(base_s, [rsplat, aidx])
            maskv = plsc.load_gather(mask_s, [rsplat, aidx])
            hv = plsc.load_gather(x_s, [rsplat, aidx])

            gamma = (P0 + P1 * s0 - P2 * s1) * tmp0v + basev + s2
            sig = 1.0 / (1.0 + jnp.exp(-gamma * invTs))
            plsc.store_scatter(out_s, [rsplat, aidx], sig)

            e3acc = e3acc + maskv * ((1.0 - 2.0 * hv) * Kv
                                     + (hv - 0.5) * (s2 + 2.0 * Kv))
            return e3acc

        e3 = lax.fori_loop(0, rpw * 16, chunk_body, jnp.zeros((16,), F32))

        lanec = jnp.minimum(lane, NCI - 1)

        def row_body(r, e1bacc):
            rsplat = jnp.broadcast_to(r, (16,))
            lTB = lanec * (2 * B2) + bvec
            c0 = plsc.load_gather(iT_s, [rsplat, lTB])
            c1 = plsc.load_gather(iT_s, [rsplat, lTB + B2])
            g0 = plsc.load_gather(xx_s, [rsplat, c0])
            g1 = plsc.load_gather(xx_s, [rsplat, c1])
            w = jnp.where(lane < NCI, g0 * g1, 0.0)
            ev = plsc.load_gather(exps_s, [rsplat])
            return e1bacc + w * ev

        e1b = lax.fori_loop(0, rpw, row_body, jnp.zeros((16,), F32))

        part_s[0, :] = e3
        part_s[1, :] = e1b * b2v
        pltpu.sync_copy(out_s, sig_out.at[pl.ds(base_row, rpw)])
        pltpu.sync_copy(part_s, part_out.at[wid])

    return sc_kernel


def kernel(X, T, C, D, S, mask, idx_ij, idx_ijF, idx_ijT, idx_iT,
           b0, b1, b2, lam, eta, nu, mu, kap, b):
    info = plsc.get_sparse_core_info()
    nw = info.num_cores * info.num_subcores
    rpw = N // nw

    scal = jnp.concatenate([b0, b1, b2, lam, eta, nu, mu, T]).reshape(1, 8)
    b2d = jnp.asarray(b, I32).reshape(1, 1)

    XX_, tmp0, base, maskb, elbd, exps = _tc_prep(
        X, C.reshape(N, N * B2), D.reshape(N, N * B2), S.reshape(N, N * B2),
        mask.reshape(N, N * B2), scal, kap, b2d)

    Ts = T[0]
    pf = jnp.concatenate([
        b1 - b0, b2 - b1, b2, kap[0],
        jnp.sum(kap, axis=1), (1.0 / Ts)[None], b2,
        jnp.zeros((6,), F32)]).astype(F32)
    pi = jnp.broadcast_to(jnp.asarray(b, I32), (16,))

    sc = _sc_make(info.num_cores, nw, rpw)
    sig, partials = sc(XX_, tmp0, base, maskb, X, exps.reshape(nw, rpw),
                       pf, pi,
                       idx_ij.reshape(N, N * KDEG * B2),
                       idx_ijF.reshape(N, N * NCOMB * B2),
                       idx_ijT.reshape(N, N * NCOMB * NT * B2),
                       idx_iT.reshape(N, NCI * 2 * B2))
    elb = elbd[0, 0] + jnp.sum(partials)
    return (elb, sig)


# speedup vs baseline: 211.4235x; 1.0387x over previous
import functools

import jax
import jax.numpy as jnp
from jax import lax
from jax.experimental import pallas as pl
from jax.experimental.pallas import tpu as pltpu
from jax.experimental.pallas import tpu_sc as plsc

N = 256
KDEG = 4
NCOMB = 4
NT = 3
NCI = 10
B2 = 2
EPS = 1e-16
F32 = jnp.float32
I32 = jnp.int32


def _tc_body(x_ref, c_ref, d_ref, s_ref, m_ref, scal_ref, kap_ref, b_ref,
             xx_out, tmp0_out, base_out, maskb_out, elbd_out, exps_out,
             pf_out, pi_out):
    X = x_ref[...]
    b0 = scal_ref[0:1, 0:1]
    b1 = scal_ref[0:1, 1:2]
    b2s = scal_ref[0:1, 2:3]
    lam = scal_ref[0:1, 3:4]
    eta = scal_ref[0:1, 4:5]
    nu = scal_ref[0:1, 5:6]
    mu = scal_ref[0:1, 6:7]
    Ts = scal_ref[0:1, 7:8]
    bb = b_ref[...]

    p_i = lax.broadcasted_iota(I32, (N * B2, N), 0)
    j_i = lax.broadcasted_iota(I32, (N * B2, N), 1)
    sel = (p_i == B2 * j_i + bb).astype(F32)
    hp = lax.Precision.HIGHEST
    C_b = jnp.dot(c_ref[...], sel, preferred_element_type=F32, precision=hp)
    D_b = jnp.dot(d_ref[...], sel, preferred_element_type=F32, precision=hp)
    S_b = jnp.dot(s_ref[...], sel, preferred_element_type=F32, precision=hp)
    M_b = jnp.dot(m_ref[...], sel, preferred_element_type=F32, precision=hp)

    r_i = lax.broadcasted_iota(I32, (N, N), 0)
    c_i = lax.broadcasted_iota(I32, (N, N), 1)
    eye = (r_i == c_i).astype(F32)
    XT = lax.dot_general(X, eye, (((0,), (0,)), ((), ())),
                         preferred_element_type=F32, precision=hp)

    X_ = 1.0 - X + EPS
    lX_ = jnp.log(X_)
    srow = jnp.sum(lX_, axis=1, keepdims=True)
    tmp0 = jnp.exp(srow - lX_)
    XX_ = X / X_
    exps = jnp.exp(srow)

    coup = lam + eta * D_b + nu * C_b + mu * S_b
    base = (4.0 * XT - 2.0) * coup + 100.0 * (M_b - 1.0)

    e1a = jnp.sum((b0 + jnp.sum(XX_, axis=1, keepdims=True) * b1) * exps)
    e2 = jnp.sum(coup * (1.0 + 4.0 * X * XT - 2.0 * (X + XT)) * M_b)
    e4 = -jnp.sum(Ts) * jnp.sum((X * jnp.log(X + EPS) + X_ * lX_) * M_b)

    xx_out[...] = XX_
    tmp0_out[...] = tmp0
    base_out[...] = base
    maskb_out[...] = M_b
    elbd_out[...] = jnp.reshape(e1a + e2 + e4, (1, 1))
    exps_out[...] = jnp.reshape(exps, (1, N))
    kapr = kap_ref[...]
    pf_out[...] = jnp.concatenate(
        [b1 - b0, b2s - b1, b2s, kapr,
         jnp.sum(kapr, axis=1, keepdims=True), 1.0 / Ts, b2s,
         jnp.zeros((1, 6), F32)], axis=1)
    pi_out[...] = jnp.broadcast_to(bb, (1, 16))


def _tc_prep(X, C2, D2, S2, M2, scal, kap, b2d):
    return pl.pallas_call(
        _tc_body,
        out_shape=[
            jax.ShapeDtypeStruct((N, N), F32),
            jax.ShapeDtypeStruct((N, N), F32),
            jax.ShapeDtypeStruct((N, N), F32),
            jax.ShapeDtypeStruct((N, N), F32),
            jax.ShapeDtypeStruct((1, 1), F32),
            jax.ShapeDtypeStruct((1, N), F32),
            jax.ShapeDtypeStruct((1, 16), F32),
            jax.ShapeDtypeStruct((1, 16), I32),
        ],
    )(X, C2, D2, S2, M2, scal, kap, b2d)


def _splat(v):
    return jnp.broadcast_to(jnp.asarray(v, I32), (16,))


def _sc_make(nc, nw, rpw):
    mesh = plsc.VectorSubcoreMesh(core_axis_name="c", subcore_axis_name="s")

    @functools.partial(
        pl.kernel,
        out_type=[
            jax.ShapeDtypeStruct((N, N), F32),
            jax.ShapeDtypeStruct((nw, 2, 16), F32),
        ],
        mesh=mesh,
        compiler_params=pltpu.CompilerParams(needs_layout_passes=False,
                                             use_tc_tiling_on_sc=True),
        scratch_types=[
            pltpu.VMEM((rpw, N), F32),
            pltpu.VMEM((rpw, N), F32),
            pltpu.VMEM((rpw, N), F32),
            pltpu.VMEM((rpw, N), F32),
            pltpu.VMEM((rpw, N), F32),
            pltpu.VMEM((rpw,), F32),
            pltpu.VMEM((1, 16), F32),
            pltpu.VMEM((1, 16), I32),
            pltpu.VMEM((rpw, N * KDEG * B2), I32),
            pltpu.VMEM((rpw, N * NCOMB * B2), I32),
            pltpu.VMEM((rpw, N * NCOMB * NT * B2), I32),
            pltpu.VMEM((rpw, NCI * 2 * B2), I32),
            pltpu.VMEM((rpw, N), F32),
            pltpu.VMEM((2, 16), F32),
        ],
    )
    def sc_kernel(xx_hbm, tmp0_hbm, base_hbm, maskb_hbm, x_hbm, exps_hbm,
                  pf_hbm, pi_hbm, ij_hbm, ijF_hbm, ijT_hbm, iT_hbm,
                  sig_out, part_out,
                  xx_s, tmp0_s, base_s, mask_s, x_s, exps_s, pf_s, pi_s,
                  ij_s, ijF_s, ijT_s, iT_s, out_s, part_s):
        wid = lax.axis_index("s") * nc + lax.axis_index("c")
        base_row = wid * rpw

        pltpu.sync_copy(xx_hbm.at[pl.ds(base_row, rpw)], xx_s)
        pltpu.sync_copy(tmp0_hbm.at[pl.ds(base_row, rpw)], tmp0_s)
        pltpu.sync_copy(base_hbm.at[pl.ds(base_row, rpw)], base_s)
        pltpu.sync_copy(maskb_hbm.at[pl.ds(base_row, rpw)], mask_s)
        pltpu.sync_copy(x_hbm.at[pl.ds(base_row, rpw)], x_s)
        pltpu.sync_copy(exps_hbm.at[0, pl.ds(base_row, rpw)], exps_s)
        pltpu.sync_copy(pf_hbm, pf_s)
        pltpu.sync_copy(pi_hbm, pi_s)
        pltpu.sync_copy(ij_hbm.at[pl.ds(base_row, rpw)], ij_s)
        pltpu.sync_copy(ijF_hbm.at[pl.ds(base_row, rpw)], ijF_s)
        pltpu.sync_copy(ijT_hbm.at[pl.ds(base_row, rpw)], ijT_s)
        pltpu.sync_copy(iT_hbm.at[pl.ds(base_row, rpw)], iT_s)

        zero16 = _splat(0)
        lane = lax.broadcasted_iota(I32, (16,), 0)
        bvec = plsc.load_gather(pi_s, [zero16, lane])
        P0 = plsc.load_gather(pf_s, [zero16, zero16])
        P1 = plsc.load_gather(pf_s, [zero16, _splat(1)])
        P2 = plsc.load_gather(pf_s, [zero16, _splat(2)])
        kapv = [plsc.load_gather(pf_s, [zero16, _splat(3 + k)])
                for k in range(KDEG)]
        Kv = plsc.load_gather(pf_s, [zero16, _splat(7)])
        invTs = plsc.load_gather(pf_s, [zero16, _splat(8)])
        b2v = plsc.load_gather(pf_s, [zero16, _splat(9)])

        def chunk_body(t, e3acc):
            r = t // 16
            a0 = (t % 16) * 16
            rsplat = jnp.broadcast_to(r, (16,))
            aidx = a0 + lane

            aKB = aidx * (KDEG * B2) + bvec
            aFB = aidx * (NCOMB * B2) + bvec
            aTB = aidx * (NCOMB * NT * B2) + bvec

            s0 = jnp.zeros((16,), F32)
            s2 = jnp.zeros((16,), F32)
            for k in range(KDEG):
                colv = plsc.load_gather(ij_s, [rsplat, aKB + k * B2])
                s0 = s0 + plsc.load_gather(xx_s, [rsplat, colv])
                gx = plsc.load_gather(x_s, [rsplat, colv])
                s2 = s2 + (4.0 * gx - 2.0) * kapv[k]

            s1 = jnp.zeros((16,), F32)
            for c in range(NCOMB):
                colF = plsc.load_gather(ijF_s, [rsplat, aFB + c * B2])
                gF = plsc.load_gather(xx_s, [rsplat, colF])
                tsum = jnp.zeros((16,), F32)
                for t_ in range(NT):
                    colT = plsc.load_gather(
                        ijT_s, [rsplat, aTB + (c * NT + t_) * B2])
                    tsum = tsum + plsc.load_gather(xx_s, [rsplat, colT])
                s1 = s1 + gF * tsum

            tmp0v = plsc.load_gather(tmp0_s, [rsplat, aidx])
            basev = plsc.load_gather(base_s, [rsplat, aidx])
            maskv = plsc.load_gather(mask_s, [rsplat, aidx])
            hv = plsc.load_gather(x_s, [rsplat, aidx])

            gamma = (P0 + P1 * s0 - P2 * s1) * tmp0v + basev + s2
            sig = 1.0 / (1.0 + jnp.exp(-gamma * invTs))
            plsc.store_scatter(out_s, [rsplat, aidx], sig)

            e3acc = e3acc + maskv * ((1.0 - 2.0 * hv) * Kv
                                     + (hv - 0.5) * (s2 + 2.0 * Kv))
            return e3acc

        e3 = lax.fori_loop(0, rpw * 16, chunk_body, jnp.zeros((16,), F32))

        lanec = jnp.minimum(lane, NCI - 1)

        def row_body(r, e1bacc):
            rsplat = jnp.broadcast_to(r, (16,))
            lTB = lanec * (2 * B2) + bvec
            c0 = plsc.load_gather(iT_s, [rsplat, lTB])
            c1 = plsc.load_gather(iT_s, [rsplat, lTB + B2])
            g0 = plsc.load_gather(xx_s, [rsplat, c0])
            g1 = plsc.load_gather(xx_s, [rsplat, c1])
            w = jnp.where(lane < NCI, g0 * g1, 0.0)
            ev = plsc.load_gather(exps_s, [rsplat])
            return e1bacc + w * ev

        e1b = lax.fori_loop(0, rpw, row_body, jnp.zeros((16,), F32))

        part_s[0, :] = e3
        part_s[1, :] = e1b * b2v
        pltpu.sync_copy(out_s, sig_out.at[pl.ds(base_row, rpw)])
        pltpu.sync_copy(part_s, part_out.at[wid])

    return sc_kernel


def kernel(X, T, C, D, S, mask, idx_ij, idx_ijF, idx_ijT, idx_iT,
           b0, b1, b2, lam, eta, nu, mu, kap, b):
    info = plsc.get_sparse_core_info()
    nw = info.num_cores * info.num_subcores
    rpw = N // nw

    scal = jnp.concatenate([b0, b1, b2, lam, eta, nu, mu, T]).reshape(1, 8)
    b2d = jnp.asarray(b, I32).reshape(1, 1)

    XX_, tmp0, base, maskb, elbd, exps, pf, pi = _tc_prep(
        X, C.reshape(N, N * B2), D.reshape(N, N * B2), S.reshape(N, N * B2),
        mask.reshape(N, N * B2), scal, kap, b2d)

    sc = _sc_make(info.num_cores, nw, rpw)
    sig, partials = sc(XX_, tmp0, base, maskb, X, exps, pf, pi,
                       idx_ij.reshape(N, N * KDEG * B2),
                       idx_ijF.reshape(N, N * NCOMB * B2),
                       idx_ijT.reshape(N, N * NCOMB * NT * B2),
                       idx_iT.reshape(N, NCI * 2 * B2))
    elb = elbd[0, 0] + jnp.sum(partials)
    return (elb, sig)


# speedup vs baseline: 254.2797x; 1.2027x over previous
import functools

import jax
import jax.numpy as jnp
from jax import lax
from jax.experimental import pallas as pl
from jax.experimental.pallas import tpu as pltpu
from jax.experimental.pallas import tpu_sc as plsc

N = 256
KDEG = 4
NCOMB = 4
NT = 3
NCI = 10
B2 = 2
EPS = 1e-16
F32 = jnp.float32
I32 = jnp.int32


def _tc_body(x_ref, c_ref, d_ref, s_ref, m_ref, scal_ref, kap_ref, b_ref,
             xx_out, tmp0_out, base_out, maskb_out, elbd_out, exps_out,
             pf_out, pi_out):
    X = x_ref[...]
    b0 = scal_ref[0:1, 0:1]
    b1 = scal_ref[0:1, 1:2]
    b2s = scal_ref[0:1, 2:3]
    lam = scal_ref[0:1, 3:4]
    eta = scal_ref[0:1, 4:5]
    nu = scal_ref[0:1, 5:6]
    mu = scal_ref[0:1, 6:7]
    Ts = scal_ref[0:1, 7:8]
    bb = b_ref[...]

    hp = lax.Precision.HIGHEST
    sel1 = (bb == 1)
    C_b = jnp.where(sel1, c_ref[:, 1, :], c_ref[:, 0, :])
    D_b = jnp.where(sel1, d_ref[:, 1, :], d_ref[:, 0, :])
    S_b = jnp.where(sel1, s_ref[:, 1, :], s_ref[:, 0, :])
    M_b = jnp.where(sel1, m_ref[:, 1, :], m_ref[:, 0, :])

    r_i = lax.broadcasted_iota(I32, (N, N), 0)
    c_i = lax.broadcasted_iota(I32, (N, N), 1)
    eye = (r_i == c_i).astype(F32)
    XT = lax.dot_general(X, eye, (((0,), (0,)), ((), ())),
                         preferred_element_type=F32, precision=hp)

    X_ = 1.0 - X + EPS
    lX_ = jnp.log(X_)
    srow = jnp.sum(lX_, axis=1, keepdims=True)
    tmp0 = jnp.exp(srow - lX_)
    XX_ = X / X_
    exps = jnp.exp(srow)

    coup = lam + eta * D_b + nu * C_b + mu * S_b
    base = (4.0 * XT - 2.0) * coup + 100.0 * (M_b - 1.0)

    e1a = jnp.sum((b0 + jnp.sum(XX_, axis=1, keepdims=True) * b1) * exps)
    e2 = jnp.sum(coup * (1.0 + 4.0 * X * XT - 2.0 * (X + XT)) * M_b)
    e4 = -jnp.sum(Ts) * jnp.sum((X * jnp.log(X + EPS) + X_ * lX_) * M_b)

    xx_out[...] = XX_
    tmp0_out[...] = tmp0
    base_out[...] = base
    maskb_out[...] = M_b
    elbd_out[...] = jnp.reshape(e1a + e2 + e4, (1, 1))
    exps_out[...] = jnp.reshape(exps, (1, N))
    kapr = kap_ref[...]
    pf_out[...] = jnp.concatenate(
        [b1 - b0, b2s - b1, b2s, kapr,
         jnp.sum(kapr, axis=1, keepdims=True), 1.0 / Ts, b2s,
         jnp.zeros((1, 6), F32)], axis=1)
    pi_out[...] = jnp.broadcast_to(bb, (1, 16))


def _tc_prep(X, C2, D2, S2, M2, scal, kap, b2d):
    return pl.pallas_call(
        _tc_body,
        out_shape=[
            jax.ShapeDtypeStruct((N, N), F32),
            jax.ShapeDtypeStruct((N, N), F32),
            jax.ShapeDtypeStruct((N, N), F32),
            jax.ShapeDtypeStruct((N, N), F32),
            jax.ShapeDtypeStruct((1, 1), F32),
            jax.ShapeDtypeStruct((1, N), F32),
            jax.ShapeDtypeStruct((1, 16), F32),
            jax.ShapeDtypeStruct((1, 16), I32),
        ],
    )(X, C2, D2, S2, M2, scal, kap, b2d)


def _splat(v):
    return jnp.broadcast_to(jnp.asarray(v, I32), (16,))


def _sc_make(nc, nw, rpw):
    mesh = plsc.VectorSubcoreMesh(core_axis_name="c", subcore_axis_name="s")

    @functools.partial(
        pl.kernel,
        out_type=[
            jax.ShapeDtypeStruct((N, N), F32),
            jax.ShapeDtypeStruct((nw, 2, 16), F32),
        ],
        mesh=mesh,
        compiler_params=pltpu.CompilerParams(needs_layout_passes=False,
                                             use_tc_tiling_on_sc=True),
        scratch_types=[
            pltpu.VMEM((rpw, N), F32),
            pltpu.VMEM((rpw, N), F32),
            pltpu.VMEM((rpw, N), F32),
            pltpu.VMEM((rpw, N), F32),
            pltpu.VMEM((rpw, N), F32),
            pltpu.VMEM((rpw,), F32),
            pltpu.VMEM((1, 16), F32),
            pltpu.VMEM((1, 16), I32),
            pltpu.VMEM((rpw, N * KDEG * B2), I32),
            pltpu.VMEM((rpw, N * NCOMB * B2), I32),
            pltpu.VMEM((rpw, N * NCOMB * NT * B2), I32),
            pltpu.VMEM((rpw, NCI * 2 * B2), I32),
            pltpu.VMEM((rpw, N), F32),
            pltpu.VMEM((2, 16), F32),
        ],
    )
    def sc_kernel(xx_hbm, tmp0_hbm, base_hbm, maskb_hbm, x_hbm, exps_hbm,
                  pf_hbm, pi_hbm, ij_hbm, ijF_hbm, ijT_hbm, iT_hbm,
                  sig_out, part_out,
                  xx_s, tmp0_s, base_s, mask_s, x_s, exps_s, pf_s, pi_s,
                  ij_s, ijF_s, ijT_s, iT_s, out_s, part_s):
        wid = lax.axis_index("s") * nc + lax.axis_index("c")
        base_row = wid * rpw

        pltpu.sync_copy(xx_hbm.at[pl.ds(base_row, rpw)], xx_s)
        pltpu.sync_copy(tmp0_hbm.at[pl.ds(base_row, rpw)], tmp0_s)
        pltpu.sync_copy(base_hbm.at[pl.ds(base_row, rpw)], base_s)
        pltpu.sync_copy(maskb_hbm.at[pl.ds(base_row, rpw)], mask_s)
        pltpu.sync_copy(x_hbm.at[pl.ds(base_row, rpw)], x_s)
        pltpu.sync_copy(exps_hbm.at[0, pl.ds(base_row, rpw)], exps_s)
        pltpu.sync_copy(pf_hbm, pf_s)
        pltpu.sync_copy(pi_hbm, pi_s)
        pltpu.sync_copy(ij_hbm.at[pl.ds(base_row, rpw)], ij_s)
        pltpu.sync_copy(ijF_hbm.at[pl.ds(base_row, rpw)], ijF_s)
        pltpu.sync_copy(ijT_hbm.at[pl.ds(base_row, rpw)], ijT_s)
        pltpu.sync_copy(iT_hbm.at[pl.ds(base_row, rpw)], iT_s)

        zero16 = _splat(0)
        lane = lax.broadcasted_iota(I32, (16,), 0)
        bvec = plsc.load_gather(pi_s, [zero16, lane])
        bvecN = bvec * N
        P0 = plsc.load_gather(pf_s, [zero16, zero16])
        P1 = plsc.load_gather(pf_s, [zero16, _splat(1)])
        P2 = plsc.load_gather(pf_s, [zero16, _splat(2)])
        kapv = [plsc.load_gather(pf_s, [zero16, _splat(3 + k)])
                for k in range(KDEG)]
        Kv = plsc.load_gather(pf_s, [zero16, _splat(7)])
        invTs = plsc.load_gather(pf_s, [zero16, _splat(8)])
        b2v = plsc.load_gather(pf_s, [zero16, _splat(9)])

        def chunk_body(t, e3acc):
            r = t // 16
            a0 = (t % 16) * 16
            rsplat = jnp.broadcast_to(r, (16,))
            aidx = a0 + lane

            aB = bvecN + aidx

            s0 = jnp.zeros((16,), F32)
            s2 = jnp.zeros((16,), F32)
            for k in range(KDEG):
                colv = plsc.load_gather(ij_s, [rsplat, aB + k * (B2 * N)])
                s0 = s0 + plsc.load_gather(xx_s, [rsplat, colv])
                gx = plsc.load_gather(x_s, [rsplat, colv])
                s2 = s2 + (4.0 * gx - 2.0) * kapv[k]

            s1 = jnp.zeros((16,), F32)
            for c in range(NCOMB):
                colF = plsc.load_gather(ijF_s, [rsplat, aB + c * (B2 * N)])
                gF = plsc.load_gather(xx_s, [rsplat, colF])
                tsum = jnp.zeros((16,), F32)
                for t_ in range(NT):
                    colT = plsc.load_gather(
                        ijT_s, [rsplat, aB + (c * NT + t_) * (B2 * N)])
                    tsum = tsum + plsc.load_gather(xx_s, [rsplat, colT])
                s1 = s1 + gF * tsum

            tmp0v = plsc.load_gather(tmp0_s, [rsplat, aidx])
            basev = plsc.load_gather(base_s, [rsplat, aidx])
            maskv = plsc.load_gather(mask_s, [rsplat, aidx])
            hv = plsc.load_gather(x_s, [rsplat, aidx])

            gamma = (P0 + P1 * s0 - P2 * s1) * tmp0v + basev + s2
            sig = 1.0 / (1.0 + jnp.exp(-gamma * invTs))
            plsc.store_scatter(out_s, [rsplat, aidx], sig)

            e3acc = e3acc + maskv * ((1.0 - 2.0 * hv) * Kv
                                     + (hv - 0.5) * (s2 + 2.0 * Kv))
            return e3acc

        e3 = lax.fori_loop(0, rpw * 16, chunk_body, jnp.zeros((16,), F32))

        lanec = jnp.minimum(lane, NCI - 1)

        def row_body(r, e1bacc):
            rsplat = jnp.broadcast_to(r, (16,))
            lTB = lanec * (2 * B2) + bvec
            c0 = plsc.load_gather(iT_s, [rsplat, lTB])
            c1 = plsc.load_gather(iT_s, [rsplat, lTB + B2])
            g0 = plsc.load_gather(xx_s, [rsplat, c0])
            g1 = plsc.load_gather(xx_s, [rsplat, c1])
            w = jnp.where(lane < NCI, g0 * g1, 0.0)
            ev = plsc.load_gather(exps_s, [rsplat])
            return e1bacc + w * ev

        e1b = lax.fori_loop(0, rpw, row_body, jnp.zeros((16,), F32))

        part_s[0, :] = e3
        part_s[1, :] = e1b * b2v
        pltpu.sync_copy(out_s, sig_out.at[pl.ds(base_row, rpw)])
        pltpu.sync_copy(part_s, part_out.at[wid])

    return sc_kernel


def kernel(X, T, C, D, S, mask, idx_ij, idx_ijF, idx_ijT, idx_iT,
           b0, b1, b2, lam, eta, nu, mu, kap, b):
    info = plsc.get_sparse_core_info()
    nw = info.num_cores * info.num_subcores
    rpw = N // nw

    scal = jnp.concatenate([b0, b1, b2, lam, eta, nu, mu, T]).reshape(1, 8)
    b2d = jnp.asarray(b, I32).reshape(1, 1)

    C3 = jnp.transpose(C, (0, 2, 1))
    D3 = jnp.transpose(D, (0, 2, 1))
    S3 = jnp.transpose(S, (0, 2, 1))
    M3 = jnp.transpose(mask, (0, 2, 1))
    XX_, tmp0, base, maskb, elbd, exps, pf, pi = _tc_prep(
        X, C3, D3, S3, M3, scal, kap, b2d)

    ij2 = jnp.transpose(idx_ij, (0, 2, 3, 1)).reshape(N, KDEG * B2 * N)
    ijF2 = jnp.transpose(idx_ijF, (0, 2, 3, 1)).reshape(N, NCOMB * B2 * N)
    ijT2 = jnp.transpose(idx_ijT, (0, 2, 3, 4, 1)).reshape(
        N, NCOMB * NT * B2 * N)
    iT2 = idx_iT.reshape(N, NCI * 2 * B2)

    sc = _sc_make(info.num_cores, nw, rpw)
    sig, partials = sc(XX_, tmp0, base, maskb, X, exps, pf, pi,
                       ij2, ijF2, ijT2, iT2)
    elb = elbd[0, 0] + jnp.sum(partials)
    return (elb, sig)


# speedup vs baseline: 269.3433x; 1.0592x over previous
import functools

import jax
import jax.numpy as jnp
from jax import lax
from jax.experimental import pallas as pl
from jax.experimental.pallas import tpu as pltpu
from jax.experimental.pallas import tpu_sc as plsc

N = 256
KDEG = 4
NCOMB = 4
NT = 3
NCI = 10
B2 = 2
EPS = 1e-16
F32 = jnp.float32
I32 = jnp.int32


def _tc_body(x_ref, c_ref, d_ref, s_ref, m_ref, scal_ref, kap_ref, b_ref,
             xx_out, tmp0_out, base_out, maskb_out, elbd_out, exps_out,
             pf_out, pi_out):
    X = x_ref[...]
    b0 = scal_ref[0:1, 0:1]
    b1 = scal_ref[0:1, 1:2]
    b2s = scal_ref[0:1, 2:3]
    lam = scal_ref[0:1, 3:4]
    eta = scal_ref[0:1, 4:5]
    nu = scal_ref[0:1, 5:6]
    mu = scal_ref[0:1, 6:7]
    Ts = scal_ref[0:1, 7:8]
    bb = b_ref[...]

    hp = lax.Precision.HIGHEST
    sel1 = (bb == 1)
    C_b = jnp.where(sel1, c_ref[:, 1, :], c_ref[:, 0, :])
    D_b = jnp.where(sel1, d_ref[:, 1, :], d_ref[:, 0, :])
    S_b = jnp.where(sel1, s_ref[:, 1, :], s_ref[:, 0, :])
    M_b = jnp.where(sel1, m_ref[:, 1, :], m_ref[:, 0, :])

    r_i = lax.broadcasted_iota(I32, (N, N), 0)
    c_i = lax.broadcasted_iota(I32, (N, N), 1)
    eye = (r_i == c_i).astype(F32)
    XT = lax.dot_general(X, eye, (((0,), (0,)), ((), ())),
                         preferred_element_type=F32, precision=hp)

    X_ = 1.0 - X + EPS
    lX_ = jnp.log(X_)
    srow = jnp.sum(lX_, axis=1, keepdims=True)
    tmp0 = jnp.exp(srow - lX_)
    XX_ = X / X_
    exps = jnp.exp(srow)

    coup = lam + eta * D_b + nu * C_b + mu * S_b
    base = (4.0 * XT - 2.0) * coup + 100.0 * (M_b - 1.0)

    e1a = jnp.sum((b0 + jnp.sum(XX_, axis=1, keepdims=True) * b1) * exps)
    e2 = jnp.sum(coup * (1.0 + 4.0 * X * XT - 2.0 * (X + XT)) * M_b)
    e4 = -jnp.sum(Ts) * jnp.sum((X * jnp.log(X + EPS) + X_ * lX_) * M_b)

    xx_out[...] = XX_
    tmp0_out[...] = tmp0
    base_out[...] = base
    maskb_out[...] = M_b
    elbd_out[...] = jnp.reshape(e1a + e2 + e4, (1, 1))
    exps_out[...] = jnp.reshape(exps, (1, N))
    kapr = kap_ref[...]
    pf_out[...] = jnp.concatenate(
        [b1 - b0, b2s - b1, b2s, kapr,
         jnp.sum(kapr, axis=1, keepdims=True), 1.0 / Ts, b2s,
         jnp.zeros((1, 6), F32)], axis=1)
    pi_out[...] = jnp.broadcast_to(bb, (1, 16))


def _tc_prep(X, C2, D2, S2, M2, scal, kap, b2d):
    return pl.pallas_call(
        _tc_body,
        out_shape=[
            jax.ShapeDtypeStruct((N, N), F32),
            jax.ShapeDtypeStruct((N, N), F32),
            jax.ShapeDtypeStruct((N, N), F32),
            jax.ShapeDtypeStruct((N, N), F32),
            jax.ShapeDtypeStruct((1, 1), F32),
            jax.ShapeDtypeStruct((1, N), F32),
            jax.ShapeDtypeStruct((1, 16), F32),
            jax.ShapeDtypeStruct((1, 16), I32),
        ],
    )(X, C2, D2, S2, M2, scal, kap, b2d)


def _splat(v):
    return jnp.broadcast_to(jnp.asarray(v, I32), (16,))


def _sc_make(nc, nw, rpw):
    mesh = plsc.VectorSubcoreMesh(core_axis_name="c", subcore_axis_name="s")

    @functools.partial(
        pl.kernel,
        out_type=[
            jax.ShapeDtypeStruct((N, N), F32),
            jax.ShapeDtypeStruct((nw, 2, 16), F32),
        ],
        mesh=mesh,
        compiler_params=pltpu.CompilerParams(needs_layout_passes=False,
                                             use_tc_tiling_on_sc=True,
                                             disable_bounds_checks=True),
        scratch_types=[
            pltpu.VMEM((rpw, N), F32),
            pltpu.VMEM((rpw, N), F32),
            pltpu.VMEM((rpw, N), F32),
            pltpu.VMEM((rpw, N), F32),
            pltpu.VMEM((rpw, N), F32),
            pltpu.VMEM((rpw,), F32),
            pltpu.VMEM((1, 16), F32),
            pltpu.VMEM((1, 16), I32),
            pltpu.VMEM((rpw, N * KDEG * B2), I32),
            pltpu.VMEM((rpw, N * NCOMB * B2), I32),
            pltpu.VMEM((rpw, N * NCOMB * NT * B2), I32),
            pltpu.VMEM((rpw, NCI * 2 * B2), I32),
            pltpu.VMEM((rpw, N), F32),
            pltpu.VMEM((2, 16), F32),
            pltpu.SemaphoreType.DMA,
        ],
    )
    def sc_kernel(xx_hbm, tmp0_hbm, base_hbm, maskb_hbm, x_hbm, exps_hbm,
                  pf_hbm, pi_hbm, ij_hbm, ijF_hbm, ijT_hbm, iT_hbm,
                  sig_out, part_out,
                  xx_s, tmp0_s, base_s, mask_s, x_s, exps_s, pf_s, pi_s,
                  ij_s, ijF_s, ijT_s, iT_s, out_s, part_s, sem):
        wid = lax.axis_index("s") * nc + lax.axis_index("c")
        base_row = wid * rpw

        rsl = pl.ds(base_row, rpw)
        descs = [
            pltpu.async_copy(pi_hbm, pi_s, sem),
            pltpu.async_copy(pf_hbm, pf_s, sem),
            pltpu.async_copy(ij_hbm.at[rsl], ij_s, sem),
            pltpu.async_copy(ijF_hbm.at[rsl], ijF_s, sem),
            pltpu.async_copy(ijT_hbm.at[rsl], ijT_s, sem),
            pltpu.async_copy(iT_hbm.at[rsl], iT_s, sem),
            pltpu.async_copy(xx_hbm.at[rsl], xx_s, sem),
            pltpu.async_copy(tmp0_hbm.at[rsl], tmp0_s, sem),
            pltpu.async_copy(base_hbm.at[rsl], base_s, sem),
            pltpu.async_copy(maskb_hbm.at[rsl], mask_s, sem),
            pltpu.async_copy(x_hbm.at[rsl], x_s, sem),
            pltpu.async_copy(exps_hbm.at[0, rsl], exps_s, sem),
        ]
        for d in descs:
            d.wait()

        zero16 = _splat(0)
        lane = lax.broadcasted_iota(I32, (16,), 0)
        bvec = pi_s[0, :]
        bsc = bvec[0]
        bN = bsc * N
        P0 = plsc.load_gather(pf_s, [zero16, zero16])
        P1 = plsc.load_gather(pf_s, [zero16, _splat(1)])
        P2 = plsc.load_gather(pf_s, [zero16, _splat(2)])
        kapv = [plsc.load_gather(pf_s, [zero16, _splat(3 + k)])
                for k in range(KDEG)]
        Kv = plsc.load_gather(pf_s, [zero16, _splat(7)])
        invTs = plsc.load_gather(pf_s, [zero16, _splat(8)])
        b2v = plsc.load_gather(pf_s, [zero16, _splat(9)])
        K2 = Kv + Kv

        e3 = jnp.zeros((16,), F32)
        for r in range(rpw):
            rsp = jnp.broadcast_to(jnp.asarray(r, I32), (16,))

            def chunk_body(t, e3acc, r=r, rsp=rsp):
                a0 = t * 16

                skx = jnp.zeros((16,), F32)
                s0 = jnp.zeros((16,), F32)
                for k in range(KDEG):
                    colv = ij_s[r, pl.ds(bN + (k * (B2 * N) + a0), 16)]
                    s0 = s0 + plsc.load_gather(xx_s, [rsp, colv])
                    gx = plsc.load_gather(x_s, [rsp, colv])
                    skx = skx + gx * kapv[k]
                s2 = 4.0 * skx - K2

                s1 = jnp.zeros((16,), F32)
                for c in range(NCOMB):
                    colF = ijF_s[r, pl.ds(bN + (c * (B2 * N) + a0), 16)]
                    gF = plsc.load_gather(xx_s, [rsp, colF])
                    tsum = jnp.zeros((16,), F32)
                    for t_ in range(NT):
                        colT = ijT_s[
                            r, pl.ds(bN + ((c * NT + t_) * (B2 * N) + a0), 16)]
                        tsum = tsum + plsc.load_gather(xx_s, [rsp, colT])
                    s1 = s1 + gF * tsum

                tmp0v = tmp0_s[r, pl.ds(a0, 16)]
                basev = base_s[r, pl.ds(a0, 16)]
                maskv = mask_s[r, pl.ds(a0, 16)]
                hv = x_s[r, pl.ds(a0, 16)]

                gamma = (P0 + P1 * s0 - P2 * s1) * tmp0v + basev + s2
                out_s[r, pl.ds(a0, 16)] = 1.0 / (1.0 + jnp.exp(-gamma * invTs))

                return e3acc + maskv * ((1.0 - 2.0 * hv) * Kv
                                        + (hv - 0.5) * (s2 + K2))

            e3 = lax.fori_loop(0, N // 16, chunk_body, e3)

        lanec = jnp.minimum(lane, NCI - 1)

        def row_body(r, e1bacc):
            rsplat = jnp.broadcast_to(r, (16,))
            lTB = lanec * (2 * B2) + bvec
            c0 = plsc.load_gather(iT_s, [rsplat, lTB])
            c1 = plsc.load_gather(iT_s, [rsplat, lTB + B2])
            g0 = plsc.load_gather(xx_s, [rsplat, c0])
            g1 = plsc.load_gather(xx_s, [rsplat, c1])
            w = jnp.where(lane < NCI, g0 * g1, 0.0)
            ev = plsc.load_gather(exps_s, [rsplat])
            return e1bacc + w * ev

        e1b = lax.fori_loop(0, rpw, row_body, jnp.zeros((16,), F32))

        part_s[0, :] = e3
        part_s[1, :] = e1b * b2v
        pltpu.sync_copy(out_s, sig_out.at[pl.ds(base_row, rpw)])
        pltpu.sync_copy(part_s, part_out.at[wid])

    return sc_kernel


def kernel(X, T, C, D, S, mask, idx_ij, idx_ijF, idx_ijT, idx_iT,
           b0, b1, b2, lam, eta, nu, mu, kap, b):
    info = plsc.get_sparse_core_info()
    nw = info.num_cores * info.num_subcores
    rpw = N // nw

    scal = jnp.concatenate([b0, b1, b2, lam, eta, nu, mu, T]).reshape(1, 8)
    b2d = jnp.asarray(b, I32).reshape(1, 1)

    C3 = jnp.transpose(C, (0, 2, 1))
    D3 = jnp.transpose(D, (0, 2, 1))
    S3 = jnp.transpose(S, (0, 2, 1))
    M3 = jnp.transpose(mask, (0, 2, 1))
    XX_, tmp0, base, maskb, elbd, exps, pf, pi = _tc_prep(
        X, C3, D3, S3, M3, scal, kap, b2d)

    ij2 = jnp.transpose(idx_ij, (0, 2, 3, 1)).reshape(N, KDEG * B2 * N)
    ijF2 = jnp.transpose(idx_ijF, (0, 2, 3, 1)).reshape(N, NCOMB * B2 * N)
    ijT2 = jnp.transpose(idx_ijT, (0, 2, 3, 4, 1)).reshape(
        N, NCOMB * NT * B2 * N)
    iT2 = idx_iT.reshape(N, NCI * 2 * B2)

    sc = _sc_make(info.num_cores, nw, rpw)
    sig, partials = sc(XX_, tmp0, base, maskb, X, exps, pf, pi,
                       ij2, ijF2, ijT2, iT2)
    elb = elbd[0, 0] + jnp.sum(partials)
    return (elb, sig)


# speedup vs baseline: 412.9006x; 1.5330x over previous
import functools

import jax
import jax.numpy as jnp
from jax import lax
from jax.experimental import pallas as pl
from jax.experimental.pallas import tpu as pltpu
from jax.experimental.pallas import tpu_sc as plsc

N = 256
KDEG = 4
NCOMB = 4
NT = 3
NCI = 10
B2 = 2
EPS = 1e-16
F32 = jnp.float32
I32 = jnp.int32


def _tc_body(x_ref, c_ref, d_ref, s_ref, m_ref, scal_ref, kap_ref, b_ref,
             xx_out, tmp0_out, base_out, maskb_out, elbd_out, exps_out,
             pf_out, pi_out):
    X = x_ref[...]
    b0 = scal_ref[0:1, 0:1]
    b1 = scal_ref[0:1, 1:2]
    b2s = scal_ref[0:1, 2:3]
    lam = scal_ref[0:1, 3:4]
    eta = scal_ref[0:1, 4:5]
    nu = scal_ref[0:1, 5:6]
    mu = scal_ref[0:1, 6:7]
    Ts = scal_ref[0:1, 7:8]
    bb = b_ref[...]

    hp = lax.Precision.HIGHEST
    sel1 = (bb == 1)
    C_b = jnp.where(sel1, c_ref[:, 1, :], c_ref[:, 0, :])
    D_b = jnp.where(sel1, d_ref[:, 1, :], d_ref[:, 0, :])
    S_b = jnp.where(sel1, s_ref[:, 1, :], s_ref[:, 0, :])
    M_b = jnp.where(sel1, m_ref[:, 1, :], m_ref[:, 0, :])

    r_i = lax.broadcasted_iota(I32, (N, N), 0)
    c_i = lax.broadcasted_iota(I32, (N, N), 1)
    eye = (r_i == c_i).astype(F32)
    XT = lax.dot_general(X, eye, (((0,), (0,)), ((), ())),
                         preferred_element_type=F32, precision=hp)

    X_ = 1.0 - X + EPS
    lX_ = jnp.log(X_)
    srow = jnp.sum(lX_, axis=1, keepdims=True)
    tmp0 = jnp.exp(srow - lX_)
    XX_ = X / X_
    exps = jnp.exp(srow)

    coup = lam + eta * D_b + nu * C_b + mu * S_b
    base = (4.0 * XT - 2.0) * coup + 100.0 * (M_b - 1.0)

    e1a = jnp.sum((b0 + jnp.sum(XX_, axis=1, keepdims=True) * b1) * exps)
    e2 = jnp.sum(coup * (1.0 + 4.0 * X * XT - 2.0 * (X + XT)) * M_b)
    e4 = -jnp.sum(Ts) * jnp.sum((X * jnp.log(X + EPS) + X_ * lX_) * M_b)

    xx_out[...] = XX_
    tmp0_out[...] = tmp0
    base_out[...] = base
    maskb_out[...] = M_b
    elbd_out[...] = jnp.reshape(e1a + e2 + e4, (1, 1))
    exps_out[...] = jnp.reshape(exps, (1, N))
    kapr = kap_ref[...]
    pf_out[...] = jnp.concatenate(
        [b1 - b0, b2s - b1, b2s, kapr,
         jnp.sum(kapr, axis=1, keepdims=True), 1.0 / Ts, b2s,
         jnp.zeros((1, 6), F32)], axis=1)
    pi_out[...] = jnp.broadcast_to(bb, (1, 16))


def _tc_prep(X, C2, D2, S2, M2, scal, kap, b2d):
    return pl.pallas_call(
        _tc_body,
        out_shape=[
            jax.ShapeDtypeStruct((N, N), F32),
            jax.ShapeDtypeStruct((N, N), F32),
            jax.ShapeDtypeStruct((N, N), F32),
            jax.ShapeDtypeStruct((N, N), F32),
            jax.ShapeDtypeStruct((1, 1), F32),
            jax.ShapeDtypeStruct((1, N), F32),
            jax.ShapeDtypeStruct((1, 16), F32),
            jax.ShapeDtypeStruct((1, 16), I32),
        ],
    )(X, C2, D2, S2, M2, scal, kap, b2d)


def _splat(v):
    return jnp.broadcast_to(jnp.asarray(v, I32), (16,))


def _sc_make(nc, nw, rpw):
    mesh = plsc.VectorSubcoreMesh(core_axis_name="c", subcore_axis_name="s")

    @functools.partial(
        pl.kernel,
        out_type=[
            jax.ShapeDtypeStruct((N, N), F32),
            jax.ShapeDtypeStruct((nw, 2, 16), F32),
        ],
        mesh=mesh,
        compiler_params=pltpu.CompilerParams(needs_layout_passes=False,
                                             use_tc_tiling_on_sc=True,
                                             disable_bounds_checks=True),
        scratch_types=[
            pltpu.VMEM((rpw, N), F32),
            pltpu.VMEM((rpw, N), F32),
            pltpu.VMEM((rpw, N), F32),
            pltpu.VMEM((rpw, N), F32),
            pltpu.VMEM((rpw, N), F32),
            pltpu.VMEM((rpw,), F32),
            pltpu.VMEM((1, 16), F32),
            pltpu.VMEM((1, 16), I32),
            pltpu.VMEM((rpw, KDEG, B2, N), I32),
            pltpu.VMEM((rpw, NCOMB, B2, N), I32),
            pltpu.VMEM((rpw, NCOMB, NT, B2, N), I32),
            pltpu.VMEM((rpw, NCI * 2 * B2), I32),
            pltpu.VMEM((rpw, N), F32),
            pltpu.VMEM((2, 16), F32),
            pltpu.SemaphoreType.DMA,
        ],
    )
    def sc_kernel(xx_hbm, tmp0_hbm, base_hbm, maskb_hbm, x_hbm, exps_hbm,
                  pf_hbm, pi_hbm, ij_hbm, ijF_hbm, ijT_hbm, iT_hbm,
                  sig_out, part_out,
                  xx_s, tmp0_s, base_s, mask_s, x_s, exps_s, pf_s, pi_s,
                  ij_s, ijF_s, ijT_s, iT_s, out_s, part_s, sem):
        wid = lax.axis_index("s") * nc + lax.axis_index("c")
        base_row = wid * rpw

        rsl = pl.ds(base_row, rpw)
        descs = [
            pltpu.async_copy(pi_hbm, pi_s, sem),
            pltpu.async_copy(pf_hbm, pf_s, sem),
            pltpu.async_copy(ij_hbm.at[rsl], ij_s, sem),
            pltpu.async_copy(ijF_hbm.at[rsl], ijF_s, sem),
            pltpu.async_copy(ijT_hbm.at[rsl], ijT_s, sem),
            pltpu.async_copy(iT_hbm.at[rsl], iT_s, sem),
            pltpu.async_copy(xx_hbm.at[rsl], xx_s, sem),
            pltpu.async_copy(tmp0_hbm.at[rsl], tmp0_s, sem),
            pltpu.async_copy(base_hbm.at[rsl], base_s, sem),
            pltpu.async_copy(maskb_hbm.at[rsl], mask_s, sem),
            pltpu.async_copy(x_hbm.at[rsl], x_s, sem),
            pltpu.async_copy(exps_hbm.at[0, rsl], exps_s, sem),
        ]
        for d in descs:
            d.wait()

        zero16 = _splat(0)
        lane = lax.broadcasted_iota(I32, (16,), 0)
        bvec = pi_s[0, :]
        bsc = bvec[0]
        bN = bsc * N
        vij = ij_s.reshape(rpw * KDEG * B2, N)
        vijF = ijF_s.reshape(rpw * NCOMB * B2, N)
        vijT = ijT_s.reshape(rpw * NCOMB * NT * B2, N)
        P0 = plsc.load_gather(pf_s, [zero16, zero16])
        P1 = plsc.load_gather(pf_s, [zero16, _splat(1)])
        P2 = plsc.load_gather(pf_s, [zero16, _splat(2)])
        kapv = [plsc.load_gather(pf_s, [zero16, _splat(3 + k)])
                for k in range(KDEG)]
        Kv = plsc.load_gather(pf_s, [zero16, _splat(7)])
        invTs = plsc.load_gather(pf_s, [zero16, _splat(8)])
        b2v = plsc.load_gather(pf_s, [zero16, _splat(9)])
        K2 = Kv + Kv

        e3 = jnp.zeros((16,), F32)
        for r in range(rpw):
            rsp = jnp.broadcast_to(jnp.asarray(r, I32), (16,))

            def chunk_body(t, e3acc, r=r, rsp=rsp):
                a0 = t * 16

                skx = jnp.zeros((16,), F32)
                s0 = jnp.zeros((16,), F32)
                for k in range(KDEG):
                    colv = vij[(r * KDEG + k) * B2 + bsc, pl.ds(a0, 16)]
                    s0 = s0 + plsc.load_gather(xx_s, [rsp, colv])
                    gx = plsc.load_gather(x_s, [rsp, colv])
                    skx = skx + gx * kapv[k]
                s2 = 4.0 * skx - K2

                s1 = jnp.zeros((16,), F32)
                for c in range(NCOMB):
                    colF = vijF[(r * NCOMB + c) * B2 + bsc, pl.ds(a0, 16)]
                    gF = plsc.load_gather(xx_s, [rsp, colF])
                    tsum = jnp.zeros((16,), F32)
                    for t_ in range(NT):
                        colT = vijT[((r * NCOMB + c) * NT + t_) * B2 + bsc,
                                    pl.ds(a0, 16)]
                        tsum = tsum + plsc.load_gather(xx_s, [rsp, colT])
                    s1 = s1 + gF * tsum

                tmp0v = tmp0_s[r, pl.ds(a0, 16)]
                basev = base_s[r, pl.ds(a0, 16)]
                maskv = mask_s[r, pl.ds(a0, 16)]
                hv = x_s[r, pl.ds(a0, 16)]

                gamma = (P0 + P1 * s0 - P2 * s1) * tmp0v + basev + s2
                out_s[r, pl.ds(a0, 16)] = 1.0 / (1.0 + jnp.exp(-gamma * invTs))

                return e3acc + maskv * ((1.0 - 2.0 * hv) * Kv
                                        + (hv - 0.5) * (s2 + K2))

            e3 = lax.fori_loop(0, N // 16, chunk_body, e3)

        lanec = jnp.minimum(lane, NCI - 1)

        def row_body(r, e1bacc):
            rsplat = jnp.broadcast_to(r, (16,))
            lTB = lanec * (2 * B2) + bvec
            c0 = plsc.load_gather(iT_s, [rsplat, lTB])
            c1 = plsc.load_gather(iT_s, [rsplat, lTB + B2])
            g0 = plsc.load_gather(xx_s, [rsplat, c0])
            g1 = plsc.load_gather(xx_s, [rsplat, c1])
            w = jnp.where(lane < NCI, g0 * g1, 0.0)
            ev = plsc.load_gather(exps_s, [rsplat])
            return e1bacc + w * ev

        e1b = lax.fori_loop(0, rpw, row_body, jnp.zeros((16,), F32))

        part_s[0, :] = e3
        part_s[1, :] = e1b * b2v
        pltpu.sync_copy(out_s, sig_out.at[pl.ds(base_row, rpw)])
        pltpu.sync_copy(part_s, part_out.at[wid])

    return sc_kernel


def kernel(X, T, C, D, S, mask, idx_ij, idx_ijF, idx_ijT, idx_iT,
           b0, b1, b2, lam, eta, nu, mu, kap, b):
    info = plsc.get_sparse_core_info()
    nw = info.num_cores * info.num_subcores
    rpw = N // nw

    scal = jnp.concatenate([b0, b1, b2, lam, eta, nu, mu, T]).reshape(1, 8)
    b2d = jnp.asarray(b, I32).reshape(1, 1)

    C3 = jnp.transpose(C, (0, 2, 1))
    D3 = jnp.transpose(D, (0, 2, 1))
    S3 = jnp.transpose(S, (0, 2, 1))
    M3 = jnp.transpose(mask, (0, 2, 1))
    XX_, tmp0, base, maskb, elbd, exps, pf, pi = _tc_prep(
        X, C3, D3, S3, M3, scal, kap, b2d)

    ij2 = jnp.transpose(idx_ij, (0, 2, 3, 1))
    ijF2 = jnp.transpose(idx_ijF, (0, 2, 3, 1))
    ijT2 = jnp.transpose(idx_ijT, (0, 2, 3, 4, 1))
    iT2 = idx_iT.reshape(N, NCI * 2 * B2)

    sc = _sc_make(info.num_cores, nw, rpw)
    sig, partials = sc(XX_, tmp0, base, maskb, X, exps, pf, pi,
                       ij2, ijF2, ijT2, iT2)
    elb = elbd[0, 0] + jnp.sum(partials)
    return (elb, sig)
